# baseline (speedup 1.0000x reference)
import functools

import jax
import jax.numpy as jnp
from jax import lax
from jax.experimental import pallas as pl
from jax.experimental.pallas import tpu as pltpu
from jax.experimental.pallas import tpu_sc as plsc

N_NODES = 10000
D = 128
NC = 2
NS = 16
EDGE_B = 128
ROW_CH = 624
ROW_SPAN = 640
N_ACC = 10000
N_DEG = 10240
_ZCHUNK = 2560


def _make_deg(nb, nb_tot):
    mesh = plsc.VectorSubcoreMesh(core_axis_name="c", subcore_axis_name="s")

    @functools.partial(
        pl.kernel, mesh=mesh,
        out_type=jax.ShapeDtypeStruct((NC, N_DEG), jnp.float32),
        scratch_types=[
            pltpu.VMEM((nb * EDGE_B,), jnp.int32),
            pltpu.VMEM((nb, EDGE_B), jnp.int32),
            pltpu.VMEM((nb * EDGE_B,), jnp.float32),
            pltpu.VMEM((_ZCHUNK,), jnp.float32),
            pltpu.VMEM_SHARED((N_DEG,), jnp.float32),
        ],
    )
    def deg_k(adj_hbm, adjf_hbm, deg_hbm, dbuf1, dbuf2, ones_v, zbuf, deg_sh):
        cid = lax.axis_index("c")
        sid = lax.axis_index("s")

        def ob(i, c):
            ones_v[pl.ds(i * 16, 16)] = jnp.ones((16,), jnp.float32)
            return c
        lax.fori_loop(0, nb * EDGE_B // 16, ob, 0)

        def zb(i, c):
            zbuf[pl.ds(i * 16, 16)] = jnp.zeros((16,), jnp.float32)
            return c
        lax.fori_loop(0, _ZCHUNK // 16, zb, 0)

        @pl.when(sid == 0)
        def _():
            def zcopy(k, c):
                pltpu.sync_copy(zbuf, deg_sh.at[pl.ds(k * _ZCHUNK, _ZCHUNK)])
                return c
            lax.fori_loop(0, N_DEG // _ZCHUNK, zcopy, 0)

        plsc.subcore_barrier()

        wid = cid * NS + sid
        nb_real = jnp.clip(nb_tot - wid * nb, 0, nb)

        @pl.when(nb_real == nb)
        def _():
            pltpu.sync_copy(adjf_hbm.at[1, cid, sid], dbuf1)
            pltpu.sync_copy(ones_v, deg_sh.at[dbuf1], add=True)

        @pl.when(nb_real < nb)
        def _():
            pltpu.sync_copy(adj_hbm.at[1, cid, sid], dbuf2)

            def body(j, c):
                pltpu.sync_copy(ones_v.at[pl.ds(0, EDGE_B)],
                                deg_sh.at[dbuf2.at[j]], add=True)
                return c
            lax.fori_loop(0, nb_real, body, 0)

        plsc.subcore_barrier()

        @pl.when(sid == 0)
        def _():
            pltpu.sync_copy(deg_sh, deg_hbm.at[cid])

    return deg_k


def _make_agg(nb, nb_tot):
    mesh = plsc.VectorSubcoreMesh(core_axis_name="c", subcore_axis_name="s")

    assert nb % 2 == 0
    hb = nb // 2
    assert hb % 2 == 0

    @functools.partial(
        pl.kernel, mesh=mesh,
        out_type=jax.ShapeDtypeStruct((NC, N_NODES, D), jnp.float32),
        scratch_types=[
            pltpu.VMEM((hb, EDGE_B), jnp.int32),
            pltpu.VMEM((hb, EDGE_B), jnp.int32),
            pltpu.VMEM((EDGE_B, D), jnp.float32),
            pltpu.VMEM((EDGE_B, D), jnp.float32),
            pltpu.VMEM_SHARED((N_ACC, D), jnp.float32),
            pltpu.SemaphoreType.DMA,
            pltpu.SemaphoreType.DMA,
        ],
    )
    def agg_k(adj_hbm, u_hbm, acc_hbm,
              sbuf, dbuf, rows_a, rows_b, acc_sh, sem_a, sem_b):
        cid = lax.axis_index("c")
        sid = lax.axis_index("s")

        def wait_g(rows, sem):
            pltpu.make_async_copy(u_hbm.at[sbuf.at[0]], rows, sem).wait()

        wid = cid * NS + sid
        nb_real = jnp.clip(nb_tot - wid * nb, 0, nb)

        pltpu.sync_copy(adj_hbm.at[0, cid, sid, pl.ds(0, hb)], sbuf)
        pltpu.sync_copy(adj_hbm.at[1, cid, sid, pl.ds(0, hb)], dbuf)

        @pl.when(nb_real > 0)
        def _():
            pltpu.async_copy(u_hbm.at[sbuf.at[0]], rows_b, sem_b)

        def zr(i, c):
            for k in range(D // 16):
                rows_a[i, pl.ds(k * 16, 16)] = jnp.zeros((16,), jnp.float32)
            return c
        lax.fori_loop(0, EDGE_B, zr, 0)

        r0 = sid * ROW_CH
        for k in range(ROW_SPAN // 80):
            pltpu.sync_copy(rows_a.at[pl.ds(0, 80)],
                            acc_sh.at[pl.ds(r0 + k * 80, 80)])
        plsc.subcore_barrier()

        for h in range(2):
            n_h = jnp.clip(nb_real - h * hb, 0, hb)
            pairs = n_h // 2

            @pl.when(jnp.logical_and(jnp.bool_(h == 1), n_h > 0))
            def _():
                pltpu.sync_copy(adj_hbm.at[0, cid, sid, pl.ds(hb, hb)], sbuf)
                pltpu.sync_copy(adj_hbm.at[1, cid, sid, pl.ds(hb, hb)], dbuf)
                pltpu.async_copy(u_hbm.at[sbuf.at[0]], rows_b, sem_b)

            def body(j, c):
                pltpu.async_copy(u_hbm.at[sbuf.at[2 * j + 1]], rows_a, sem_a)
                wait_g(rows_b, sem_b)
                pltpu.sync_copy(rows_b, acc_sh.at[dbuf.at[2 * j]], add=True)

                @pl.when(j < pairs - 1)
                def _():
                    pltpu.async_copy(u_hbm.at[sbuf.at[2 * j + 2]], rows_b, sem_b)
                wait_g(rows_a, sem_a)
                pltpu.sync_copy(rows_a, acc_sh.at[dbuf.at[2 * j + 1]], add=True)
                return c
            lax.fori_loop(0, pairs, body, 0)

        plsc.subcore_barrier()
        pltpu.sync_copy(acc_sh.at[pl.ds(r0, ROW_SPAN)],
                        acc_hbm.at[cid, pl.ds(r0, ROW_SPAN)])

    return agg_k


_BM = 1024
_GRID = -(-N_NODES // _BM)


def _deg_col(dp):
    ones21 = jnp.ones((2, 1), jnp.float32)
    return lax.dot_general(dp, ones21, (((0,), (0,)), ((), ()))) + 1.0


def _mm_body(x_ref, w_ref, h_ref):
    h_ref[...] = jnp.dot(x_ref[...].astype(jnp.bfloat16),
                         w_ref[...].astype(jnp.bfloat16),
                         preferred_element_type=jnp.float32)


def _scale_body(h_ref, dp_ref, u_ref):
    dis = lax.rsqrt(_deg_col(dp_ref[...]))
    u_ref[...] = h_ref[...] * dis


def _fin_body(acc_ref, u_ref, dp_ref, b_ref, o_ref):
    dis = lax.rsqrt(_deg_col(dp_ref[...]))
    s = acc_ref[0] + acc_ref[1] + u_ref[...]
    o_ref[...] = s * dis + b_ref[...]


_mm = pl.pallas_call(
    _mm_body,
    grid=(_GRID,),
    in_specs=[
        pl.BlockSpec((_BM, D), lambda i: (i, 0)),
        pl.BlockSpec((D, D), lambda i: (0, 0)),
    ],
    out_specs=pl.BlockSpec((_BM, D), lambda i: (i, 0)),
    out_shape=jax.ShapeDtypeStruct((N_NODES, D), jnp.float32),
)

_scale = pl.pallas_call(
    _scale_body,
    grid=(_GRID,),
    in_specs=[
        pl.BlockSpec((_BM, D), lambda i: (i, 0)),
        pl.BlockSpec((NC, _BM), lambda i: (0, i)),
    ],
    out_specs=pl.BlockSpec((_BM, D), lambda i: (i, 0)),
    out_shape=jax.ShapeDtypeStruct((N_NODES, D), jnp.float32),
)

_fin = pl.pallas_call(
    _fin_body,
    grid=(_GRID,),
    in_specs=[
        pl.BlockSpec((NC, _BM, D), lambda i: (0, i, 0)),
        pl.BlockSpec((_BM, D), lambda i: (i, 0)),
        pl.BlockSpec((NC, _BM), lambda i: (0, i)),
        pl.BlockSpec((1, D), lambda i: (0, 0)),
    ],
    out_specs=pl.BlockSpec((_BM, D), lambda i: (i, 0)),
    out_shape=jax.ShapeDtypeStruct((N_NODES, D), jnp.float32),
)


def kernel(x, adj_t, W, b):
    E = adj_t.shape[1]
    assert E % (2 * EDGE_B) == 0
    nb_tot = E // EDGE_B
    chunk = NC * NS * EDGE_B
    nb2 = -(-E // chunk)
    nb2 += nb2 % 2
    e_pad = nb2 * chunk - E
    if e_pad:
        adj_t = jnp.pad(adj_t, ((0, 0), (0, e_pad)))
    nb = nb2
    adj = adj_t.reshape(2, NC, NS, nb, EDGE_B)
    adjf = adj_t.reshape(2, NC, NS, nb * EDGE_B)

    h = _mm(x, W)
    degp = _make_deg(nb, nb_tot)(adj, adjf)
    u = _scale(h, degp)
    accs = _make_agg(nb, nb_tot)(adj, u)
    return _fin(accs, u, degp, b.reshape(1, D))


# speedup vs baseline: 1.0100x; 1.0100x over previous
import functools

import jax
import jax.numpy as jnp
from jax import lax
from jax.experimental import pallas as pl
from jax.experimental.pallas import tpu as pltpu
from jax.experimental.pallas import tpu_sc as plsc

N_NODES = 10000
D = 128
NC = 2
NS = 16
EDGE_B = 128
ROW_CH = 624
ROW_SPAN = 640
N_ACC = 10000
N_DEG = 10240
_ZCHUNK = 2560


def _make_deg(nb, nb_tot):
    mesh = plsc.VectorSubcoreMesh(core_axis_name="c", subcore_axis_name="s")

    @functools.partial(
        pl.kernel, mesh=mesh,
        out_type=jax.ShapeDtypeStruct((NC, N_DEG), jnp.float32),
        scratch_types=[
            pltpu.VMEM((nb, EDGE_B), jnp.int32),
            pltpu.VMEM((EDGE_B,), jnp.float32),
            pltpu.VMEM((_ZCHUNK,), jnp.float32),
            pltpu.VMEM_SHARED((N_DEG,), jnp.float32),
        ],
    )
    def deg_k(adj_hbm, deg_hbm, dbuf, ones_v, zbuf, deg_sh):
        cid = lax.axis_index("c")
        sid = lax.axis_index("s")
        for k in range(EDGE_B // 16):
            ones_v[pl.ds(k * 16, 16)] = jnp.ones((16,), jnp.float32)

        def zb(i, c):
            zbuf[pl.ds(i * 16, 16)] = jnp.zeros((16,), jnp.float32)
            return c
        lax.fori_loop(0, _ZCHUNK // 16, zb, 0)

        @pl.when(sid == 0)
        def _():
            def zcopy(k, c):
                pltpu.sync_copy(zbuf, deg_sh.at[pl.ds(k * _ZCHUNK, _ZCHUNK)])
                return c
            lax.fori_loop(0, N_DEG // _ZCHUNK, zcopy, 0)

        plsc.subcore_barrier()
        pltpu.sync_copy(adj_hbm.at[1, cid, sid], dbuf)

        wid = cid * NS + sid
        nb_real = jnp.clip(nb_tot - wid * nb, 0, nb)

        def body(j, c):
            pltpu.sync_copy(ones_v, deg_sh.at[dbuf.at[j]], add=True)
            return c
        lax.fori_loop(0, nb_real, body, 0)

        plsc.subcore_barrier()

        @pl.when(sid == 0)
        def _():
            pltpu.sync_copy(deg_sh, deg_hbm.at[cid])

    return deg_k


def _make_agg(nb, nb_tot):
    mesh = plsc.VectorSubcoreMesh(core_axis_name="c", subcore_axis_name="s")

    assert nb % 2 == 0
    hb = nb // 2
    assert hb % 2 == 0

    @functools.partial(
        pl.kernel, mesh=mesh,
        out_type=jax.ShapeDtypeStruct((NC, N_NODES, D), jnp.float32),
        scratch_types=[
            pltpu.VMEM((hb, EDGE_B), jnp.int32),
            pltpu.VMEM((hb, EDGE_B), jnp.int32),
            pltpu.VMEM((EDGE_B, D), jnp.float32),
            pltpu.VMEM((EDGE_B, D), jnp.float32),
            pltpu.VMEM_SHARED((N_ACC, D), jnp.float32),
            pltpu.SemaphoreType.DMA,
            pltpu.SemaphoreType.DMA,
        ],
    )
    def agg_k(adj_hbm, u_hbm, acc_hbm,
              sbuf, dbuf, rows_a, rows_b, acc_sh, sem_a, sem_b):
        cid = lax.axis_index("c")
        sid = lax.axis_index("s")

        def wait_g(rows, sem):
            pltpu.make_async_copy(u_hbm.at[sbuf.at[0]], rows, sem).wait()

        wid = cid * NS + sid
        nb_real = jnp.clip(nb_tot - wid * nb, 0, nb)

        pltpu.sync_copy(adj_hbm.at[0, cid, sid, pl.ds(0, hb)], sbuf)
        pltpu.sync_copy(adj_hbm.at[1, cid, sid, pl.ds(0, hb)], dbuf)

        @pl.when(nb_real > 0)
        def _():
            pltpu.async_copy(u_hbm.at[sbuf.at[0]], rows_b, sem_b)

        def zr(i, c):
            for k in range(D // 16):
                rows_a[i, pl.ds(k * 16, 16)] = jnp.zeros((16,), jnp.float32)
            return c
        lax.fori_loop(0, EDGE_B, zr, 0)

        r0 = sid * ROW_CH
        for k in range(ROW_SPAN // 80):
            pltpu.sync_copy(rows_a.at[pl.ds(0, 80)],
                            acc_sh.at[pl.ds(r0 + k * 80, 80)])
        plsc.subcore_barrier()

        for h in range(2):
            n_h = jnp.clip(nb_real - h * hb, 0, hb)
            pairs = n_h // 2

            @pl.when(jnp.logical_and(jnp.bool_(h == 1), n_h > 0))
            def _():
                pltpu.sync_copy(adj_hbm.at[0, cid, sid, pl.ds(hb, hb)], sbuf)
                pltpu.sync_copy(adj_hbm.at[1, cid, sid, pl.ds(hb, hb)], dbuf)
                pltpu.async_copy(u_hbm.at[sbuf.at[0]], rows_b, sem_b)

            def body(j, c):
                pltpu.async_copy(u_hbm.at[sbuf.at[2 * j + 1]], rows_a, sem_a)
                wait_g(rows_b, sem_b)
                pltpu.sync_copy(rows_b, acc_sh.at[dbuf.at[2 * j]], add=True)

                @pl.when(j < pairs - 1)
                def _():
                    pltpu.async_copy(u_hbm.at[sbuf.at[2 * j + 2]], rows_b, sem_b)
                wait_g(rows_a, sem_a)
                pltpu.sync_copy(rows_a, acc_sh.at[dbuf.at[2 * j + 1]], add=True)
                return c
            lax.fori_loop(0, pairs, body, 0)

        plsc.subcore_barrier()
        pltpu.sync_copy(acc_sh.at[pl.ds(r0, ROW_SPAN)],
                        acc_hbm.at[cid, pl.ds(r0, ROW_SPAN)])

    return agg_k


_BM = 1024
_GRID = -(-N_NODES // _BM)


def _deg_col(dp):
    ones21 = jnp.ones((2, 1), jnp.float32)
    return lax.dot_general(dp, ones21, (((0,), (0,)), ((), ()))) + 1.0


def _lin_body(x_ref, w_ref, dp_ref, u_ref):
    dis = lax.rsqrt(_deg_col(dp_ref[...]))
    h = jnp.dot(x_ref[...], w_ref[...], preferred_element_type=jnp.float32)
    u_ref[...] = h * dis


def _fin_body(acc_ref, u_ref, dp_ref, b_ref, o_ref):
    dis = lax.rsqrt(_deg_col(dp_ref[...]))
    s = acc_ref[0] + acc_ref[1] + u_ref[...]
    o_ref[...] = s * dis + b_ref[...]


_lin = pl.pallas_call(
    _lin_body,
    grid=(_GRID,),
    in_specs=[
        pl.BlockSpec((_BM, D), lambda i: (i, 0)),
        pl.BlockSpec((D, D), lambda i: (0, 0)),
        pl.BlockSpec((NC, _BM), lambda i: (0, i)),
    ],
    out_specs=pl.BlockSpec((_BM, D), lambda i: (i, 0)),
    out_shape=jax.ShapeDtypeStruct((N_NODES, D), jnp.float32),
)

_fin = pl.pallas_call(
    _fin_body,
    grid=(_GRID,),
    in_specs=[
        pl.BlockSpec((NC, _BM, D), lambda i: (0, i, 0)),
        pl.BlockSpec((_BM, D), lambda i: (i, 0)),
        pl.BlockSpec((NC, _BM), lambda i: (0, i)),
        pl.BlockSpec((1, D), lambda i: (0, 0)),
    ],
    out_specs=pl.BlockSpec((_BM, D), lambda i: (i, 0)),
    out_shape=jax.ShapeDtypeStruct((N_NODES, D), jnp.float32),
)


def kernel(x, adj_t, W, b):
    E = adj_t.shape[1]
    assert E % (2 * EDGE_B) == 0
    nb_tot = E // EDGE_B
    chunk = NC * NS * EDGE_B
    nb2 = -(-E // chunk)
    nb2 += nb2 % 2
    e_pad = nb2 * chunk - E
    if e_pad:
        adj_t = jnp.pad(adj_t, ((0, 0), (0, e_pad)))
    nb = nb2
    adj = adj_t.reshape(2, NC, NS, nb, EDGE_B)

    degp = _make_deg(nb, nb_tot)(adj)
    u = _lin(x, W, degp)
    accs = _make_agg(nb, nb_tot)(adj, u)
    return _fin(accs, u, degp, b.reshape(1, D))
